# baseline (speedup 1.0000x reference)
import functools

import jax
import jax.numpy as jnp
from jax import lax
from jax.experimental import pallas as pl
from jax.experimental.pallas import tpu as pltpu
from jax.experimental.pallas import tpu_sc as plsc

_NG, _NM = 65160, 40962
_NGP, _NMP = 65536, 41472
_EG, _EM, _ED = 300000, 250000, 195480
_EGP, _EMP, _EDP = 311296, 262144, 196608
_BIGI = 1 << 29
_C = 12288
_CH = 128
_CHS = 64


def _pad_rows(x, n):
    return jnp.pad(x, ((0, n - x.shape[0]), (0, 0)))


def _pad_idx(x, n, fill):
    return jnp.pad(x.astype(jnp.int32), (0, n - x.shape[0]), constant_values=fill)


def _tc_mlp(xs, w1s, mp, ln=True, res=None, block=512):
    rows = xs[0].shape[0]
    assert rows % block == 0, (rows, block)
    nx = len(xs)
    nres = 1 if res is not None else 0
    b1 = mp["b1"].reshape(1, -1)
    b2 = mp["b2"].reshape(1, -1)
    wparams = [b1, mp["w2"], b2]
    if ln:
        wparams += [mp["g"].reshape(1, -1), mp["bln"].reshape(1, -1)]
    ops = list(xs) + list(w1s) + wparams + ([res] if res is not None else [])
    dout = mp["w2"].shape[1]

    def body(*refs):
        xr = refs[:nx]
        w1r = refs[nx:2 * nx]
        pos = 2 * nx
        b1_r, w2_r, b2_r = refs[pos], refs[pos + 1], refs[pos + 2]
        pos += 3
        if ln:
            g_r, bln_r = refs[pos], refs[pos + 1]
            pos += 2
        res_r = refs[pos:pos + nres]
        out_r = refs[pos + nres]
        h = jnp.dot(xr[0][...], w1r[0][...], preferred_element_type=jnp.float32)
        for i in range(1, nx):
            h = h + jnp.dot(xr[i][...], w1r[i][...],
                            preferred_element_type=jnp.float32)
        h = h + b1_r[...]
        h = h * jax.nn.sigmoid(h)
        o = jnp.dot(h, w2_r[...], preferred_element_type=jnp.float32) + b2_r[...]
        if ln:
            m = jnp.mean(o, axis=-1, keepdims=True)
            v = jnp.mean((o - m) * (o - m), axis=-1, keepdims=True)
            o = (o - m) * lax.rsqrt(v + 1e-5) * g_r[...] + bln_r[...]
        if nres:
            o = o + res_r[0][...]
        out_r[...] = o

    in_specs = []
    for x in xs:
        in_specs.append(pl.BlockSpec((block, x.shape[1]), lambda i: (i, 0)))
    for w in list(w1s) + wparams:
        in_specs.append(pl.BlockSpec(w.shape, lambda i: (0, 0)))
    if res is not None:
        in_specs.append(pl.BlockSpec((block, res.shape[1]), lambda i: (i, 0)))
    out = pl.pallas_call(
        body,
        grid=(rows // block,),
        in_specs=in_specs,
        out_specs=pl.BlockSpec((block, dout), lambda i: (i, 0)),
        out_shape=jax.ShapeDtypeStruct((rows, dout), jnp.float32),
    )(*ops)
    return out


def _sc_mesh():
    return plsc.VectorSubcoreMesh(core_axis_name="c", subcore_axis_name="s")


def _sc_gather2(table_a, idx_a, table_b, idx_b):
    ep = idx_a.shape[0]
    d = table_a.shape[1]
    bpw = ep // 32
    rb = 2 * _CH
    nch = bpw // rb
    assert bpw % rb == 0 and nch % 2 == 0

    @functools.partial(
        pl.kernel,
        mesh=_sc_mesh(),
        out_type=[jax.ShapeDtypeStruct((ep, d), jnp.float32)] * 2,
        scratch_types=[
            pltpu.VMEM((bpw,), jnp.int32),
            pltpu.VMEM((rb, d), jnp.float32),
            pltpu.VMEM((rb, d), jnp.float32),
            pltpu.SemaphoreType.DMA,
            pltpu.SemaphoreType.DMA,
            pltpu.SemaphoreType.DMA,
            pltpu.SemaphoreType.DMA,
        ],
    )
    def k(ta, ia, tb, ib, oa, ob, idx_v, buf0, buf1, g0, g1, w0, w1):
        wid = lax.axis_index("s") * 2 + lax.axis_index("c")
        base = wid * bpw
        bufs = (buf0, buf1)
        gsems = (g0, g1)
        wsems = (w0, w1)

        for t_hbm, i_hbm, o_hbm in ((ta, ia, oa), (tb, ib, ob)):
            pltpu.sync_copy(i_hbm.at[pl.ds(base, bpw)], idx_v)

            def startg(ci, b):
                for kk in range(2):
                    pltpu.async_copy(
                        t_hbm.at[idx_v.at[pl.ds(ci * rb + kk * _CH, _CH)]],
                        bufs[b].at[pl.ds(kk * _CH, _CH)], gsems[b])

            def waitg(ci, b):
                for kk in range(2):
                    pltpu.make_async_copy(
                        t_hbm.at[idx_v.at[pl.ds(ci * rb + kk * _CH, _CH)]],
                        bufs[b].at[pl.ds(kk * _CH, _CH)], gsems[b]).wait()

            startg(0, 0)
            startg(1, 1)

            def group(g, carry):
                for b in range(2):
                    ci = g * 2 + b
                    waitg(ci, b)
                    pltpu.async_copy(bufs[b],
                                     o_hbm.at[pl.ds(base + ci * rb, rb)],
                                     wsems[b])
                    pltpu.make_async_copy(
                        bufs[b], o_hbm.at[pl.ds(base + ci * rb, rb)],
                        wsems[b]).wait()
                    nci = ci + 2

                    @pl.when(nci < nch)
                    def _():
                        startg(nci, b)
                return carry

            lax.fori_loop(0, nch // 2, group, 0)

    return k(table_a, idx_a, table_b, idx_b)


def _sc_segment_sum(msg, adjc, np_rows):
    ep = msg.shape[0]
    nchunks = adjc.shape[0] // ep
    assert nchunks % 2 == 0
    cps = nchunks // 2
    ept = ep // 16
    nch = ept // _CHS
    assert ept % _CHS == 0 and nch % 2 == 0
    tpr = _C // 16
    nz = tpr // _CH
    assert tpr % _CH == 0

    scratch = [
        pltpu.VMEM((_CHS, 128), jnp.float32),
        pltpu.VMEM((_CHS, 128), jnp.float32),
        pltpu.VMEM((_CHS,), jnp.int32),
        pltpu.VMEM((_CHS,), jnp.int32),
        pltpu.VMEM_SHARED((_C + 8, 128), jnp.float32),
        pltpu.SemaphoreType.DMA,
        pltpu.SemaphoreType.DMA,
        pltpu.SemaphoreType.DMA,
        pltpu.SemaphoreType.DMA,
    ]

    @functools.partial(
        pl.kernel, mesh=_sc_mesh(),
        out_type=jax.ShapeDtypeStruct((np_rows, 128), jnp.float32),
        scratch_types=scratch)
    def k(msg_hbm, adj_hbm, zref_hbm, out_hbm, mb0, mb1, ib0, ib1,
          shared, sem0, sem1, ss0, ss1):
        c = lax.axis_index("c")
        s = lax.axis_index("s")
        tbase = s * ept
        mbufs = (mb0, mb1)
        ibufs = (ib0, ib1)
        sems = (sem0, sem1)
        ssems = (ss0, ss1)

        for jj in range(cps):
            chunk = c * cps + jj
            abase = chunk * ep + tbase

            def zs(z, carry):
                pltpu.sync_copy(zref_hbm,
                                shared.at[pl.ds(s * tpr + z * _CH, _CH)])
                return carry

            lax.fori_loop(0, nz, zs, 0)

            @pl.when(s == 0)
            def _():
                pltpu.sync_copy(zref_hbm.at[pl.ds(0, 8)],
                                shared.at[pl.ds(_C, 8)])

            plsc.subcore_barrier()

            def start(ci, b):
                pltpu.async_copy(msg_hbm.at[pl.ds(tbase + ci * _CHS, _CHS)],
                                 mbufs[b], sems[b])
                pltpu.async_copy(adj_hbm.at[pl.ds(abase + ci * _CHS, _CHS)],
                                 ibufs[b], sems[b])

            def wait(ci, b):
                pltpu.make_async_copy(
                    msg_hbm.at[pl.ds(tbase + ci * _CHS, _CHS)], mbufs[b],
                    sems[b]).wait()
                pltpu.make_async_copy(
                    adj_hbm.at[pl.ds(abase + ci * _CHS, _CHS)], ibufs[b],
                    sems[b]).wait()

            start(0, 0)
            start(1, 1)

            def group(g, carry):
                for b in range(2):
                    ci = g * 2 + b
                    wait(ci, b)
                    pltpu.async_copy(mbufs[b], shared.at[ibufs[b]], ssems[b],
                                     add=True)
                    pltpu.make_async_copy(mbufs[b], shared.at[ibufs[b]],
                                          ssems[b]).wait()
                    nci = ci + 2

                    @pl.when(nci < nch)
                    def _():
                        start(nci, b)
                return carry

            lax.fori_loop(0, nch // 2, group, 0)
            plsc.subcore_barrier()

            def co(z, carry):
                r0 = s * tpr + z * _CHS
                g0 = chunk * _C + r0

                @pl.when(g0 + _CHS <= np_rows)
                def _():
                    pltpu.sync_copy(shared.at[pl.ds(r0, _CHS)], mb0)
                    pltpu.sync_copy(mb0, out_hbm.at[pl.ds(g0, _CHS)])
                return carry

            lax.fori_loop(0, tpr // _CHS, co, 0)
            plsc.subcore_barrier()

    return k(msg, adjc, jnp.zeros((_CH, 128), jnp.float32))


def _scatter_rows(idx, np_rows):
    nchunks = -(-np_rows // _C)
    nchunks += nchunks % 2
    rows = [jnp.where((idx >= j * _C) & (idx < (j + 1) * _C), idx - j * _C, _C)
            for j in range(nchunks)]
    return jnp.concatenate(rows).astype(jnp.int32)


def kernel(grid_x, grid_struct, mesh_struct, g2m_edge_attr, m2m_edge_attr,
           m2g_edge_attr, g2m_senders, g2m_receivers, m2m_senders,
           m2m_receivers, m2g_senders, m2g_receivers, params):
    p = params
    gx = _pad_rows(grid_x, _NGP)
    gs = jnp.pad(grid_struct, ((0, _NGP - _NG), (0, 5)))
    ms = jnp.pad(mesh_struct, ((0, _NMP - _NM), (0, 5)))
    ea_g = jnp.pad(g2m_edge_attr, ((0, _EGP - _EG), (0, 4)))
    ea_m = jnp.pad(m2m_edge_attr, ((0, _EMP - _EM), (0, 4)))
    ea_d = jnp.pad(m2g_edge_attr, ((0, _EDP - _ED), (0, 4)))

    sg = _pad_idx(g2m_senders, _EGP, 0)
    rg_g = _pad_idx(g2m_receivers, _EGP, 0)
    rg_s = _scatter_rows(_pad_idx(g2m_receivers, _EGP, _BIGI), _NMP)
    sm = _pad_idx(m2m_senders, _EMP, 0)
    rm_g = _pad_idx(m2m_receivers, _EMP, 0)
    rm_s = _scatter_rows(_pad_idx(m2m_receivers, _EMP, _BIGI), _NMP)
    sd = _pad_idx(m2g_senders, _EDP, 0)
    rd_g = _pad_idx(m2g_receivers, _EDP, 0)
    rd_s = _scatter_rows(_pad_idx(m2g_receivers, _EDP, _BIGI), _NGP)

    mp = p["grid_embed"]
    w1 = mp["w1"]
    grid = _tc_mlp([gx, gs], [w1[:128], jnp.pad(w1[128:], ((0, 5), (0, 0)))], mp)
    mp = p["mesh_embed"]
    mesh = _tc_mlp([ms], [jnp.pad(mp["w1"], ((0, 5), (0, 0)))], mp)
    mp = p["g2m_edge_embed"]
    eg = _tc_mlp([ea_g], [jnp.pad(mp["w1"], ((0, 4), (0, 0)))], mp)
    mp = p["m2m_edge_embed"]
    em = _tc_mlp([ea_m], [jnp.pad(mp["w1"], ((0, 4), (0, 0)))], mp)
    mp = p["m2g_edge_embed"]
    ed = _tc_mlp([ea_d], [jnp.pad(mp["w1"], ((0, 4), (0, 0)))], mp)

    snd, rcv = _sc_gather2(grid, sg, mesh, rg_g)
    mp = p["g2m_edge"]
    w1 = mp["w1"]
    msg = _tc_mlp([eg, snd, rcv], [w1[:128], w1[128:256], w1[256:384]], mp)
    agg = _sc_segment_sum(msg, rg_s, _NMP)
    mp = p["g2m_node"]
    w1 = mp["w1"]
    mesh = _tc_mlp([mesh, agg], [w1[:128], w1[128:256]], mp, res=mesh)
    mp = p["g2m_grid"]
    grid = _tc_mlp([grid], [mp["w1"]], mp, res=grid)

    for pe, pn in zip(p["proc_edge"], p["proc_node"]):
        snd, rcv = _sc_gather2(mesh, sm, mesh, rm_g)
        w1 = pe["w1"]
        em = _tc_mlp([em, snd, rcv], [w1[:128], w1[128:256], w1[256:384]], pe,
                     res=em)
        agg = _sc_segment_sum(em, rm_s, _NMP)
        w1 = pn["w1"]
        mesh = _tc_mlp([mesh, agg], [w1[:128], w1[128:256]], pn, res=mesh)

    snd, rcv = _sc_gather2(mesh, sd, grid, rd_g)
    mp = p["m2g_edge"]
    w1 = mp["w1"]
    msg = _tc_mlp([ed, snd, rcv], [w1[:128], w1[128:256], w1[256:384]], mp)
    agg = _sc_segment_sum(msg, rd_s, _NGP)
    mp = p["m2g_node"]
    w1 = mp["w1"]
    grid = _tc_mlp([grid, agg], [w1[:128], w1[128:256]], mp, res=grid)
    mp = p["out"]
    out = _tc_mlp([grid], [mp["w1"]], mp, ln=False, res=gx)
    return out[:_NG]


# speedup vs baseline: 1.0285x; 1.0285x over previous
import functools

import jax
import jax.numpy as jnp
from jax import lax
from jax.experimental import pallas as pl
from jax.experimental.pallas import tpu as pltpu
from jax.experimental.pallas import tpu_sc as plsc

_NG, _NM = 65160, 40962
_NGP, _NMP = 65536, 41472
_EG, _EM, _ED = 300000, 250000, 195480
_EGP, _EMP, _EDP = 311296, 262144, 196608
_BIGI = 1 << 29
_C = 12288
_CH = 128
_CHS = 64


def _pad_rows(x, n):
    return jnp.pad(x, ((0, n - x.shape[0]), (0, 0)))


def _pad_idx(x, n, fill):
    return jnp.pad(x.astype(jnp.int32), (0, n - x.shape[0]), constant_values=fill)


def _tc_mlp(xs, w1s, mp, ln=True, res=None, block=512):
    rows = xs[0].shape[0]
    assert rows % block == 0, (rows, block)
    nx = len(xs)
    nres = 1 if res is not None else 0
    b1 = mp["b1"].reshape(1, -1)
    b2 = mp["b2"].reshape(1, -1)
    wparams = [b1, mp["w2"], b2]
    if ln:
        wparams += [mp["g"].reshape(1, -1), mp["bln"].reshape(1, -1)]
    ops = list(xs) + list(w1s) + wparams + ([res] if res is not None else [])
    dout = mp["w2"].shape[1]

    def body(*refs):
        xr = refs[:nx]
        w1r = refs[nx:2 * nx]
        pos = 2 * nx
        b1_r, w2_r, b2_r = refs[pos], refs[pos + 1], refs[pos + 2]
        pos += 3
        if ln:
            g_r, bln_r = refs[pos], refs[pos + 1]
            pos += 2
        res_r = refs[pos:pos + nres]
        out_r = refs[pos + nres]
        h = jnp.dot(xr[0][...], w1r[0][...], preferred_element_type=jnp.float32)
        for i in range(1, nx):
            h = h + jnp.dot(xr[i][...], w1r[i][...],
                            preferred_element_type=jnp.float32)
        h = h + b1_r[...]
        h = h * jax.nn.sigmoid(h)
        o = jnp.dot(h, w2_r[...], preferred_element_type=jnp.float32) + b2_r[...]
        if ln:
            m = jnp.mean(o, axis=-1, keepdims=True)
            v = jnp.mean((o - m) * (o - m), axis=-1, keepdims=True)
            o = (o - m) * lax.rsqrt(v + 1e-5) * g_r[...] + bln_r[...]
        if nres:
            o = o + res_r[0][...]
        out_r[...] = o

    in_specs = []
    for x in xs:
        in_specs.append(pl.BlockSpec((block, x.shape[1]), lambda i: (i, 0)))
    for w in list(w1s) + wparams:
        in_specs.append(pl.BlockSpec(w.shape, lambda i: (0, 0)))
    if res is not None:
        in_specs.append(pl.BlockSpec((block, res.shape[1]), lambda i: (i, 0)))
    out = pl.pallas_call(
        body,
        grid=(rows // block,),
        in_specs=in_specs,
        out_specs=pl.BlockSpec((block, dout), lambda i: (i, 0)),
        out_shape=jax.ShapeDtypeStruct((rows, dout), jnp.float32),
    )(*ops)
    return out


def _sc_mesh():
    return plsc.VectorSubcoreMesh(core_axis_name="c", subcore_axis_name="s")


def _sc_gather(table, idx):
    ep = idx.shape[0]
    d = table.shape[1]
    bpw = ep // 32
    rb = 2 * _CH
    nch = bpw // rb
    assert bpw % rb == 0 and nch % 2 == 0

    @functools.partial(
        pl.kernel,
        mesh=_sc_mesh(),
        out_type=jax.ShapeDtypeStruct((ep, d), jnp.float32),
        scratch_types=[
            pltpu.VMEM((bpw,), jnp.int32),
            pltpu.VMEM((rb, d), jnp.float32),
            pltpu.VMEM((rb, d), jnp.float32),
            pltpu.SemaphoreType.DMA,
            pltpu.SemaphoreType.DMA,
        ],
    )
    def k(t_hbm, i_hbm, o_hbm, idx_v, buf0, buf1, g0, g1):
        wid = lax.axis_index("s") * 2 + lax.axis_index("c")
        base = wid * bpw
        bufs = (buf0, buf1)
        gsems = (g0, g1)
        pltpu.sync_copy(i_hbm.at[pl.ds(base, bpw)], idx_v)

        def startg(ci, b):
            for kk in range(2):
                pltpu.async_copy(
                    t_hbm.at[idx_v.at[pl.ds(ci * rb + kk * _CH, _CH)]],
                    bufs[b].at[pl.ds(kk * _CH, _CH)], gsems[b])

        def waitg(ci, b):
            for kk in range(2):
                pltpu.make_async_copy(
                    t_hbm.at[idx_v.at[pl.ds(ci * rb + kk * _CH, _CH)]],
                    bufs[b].at[pl.ds(kk * _CH, _CH)], gsems[b]).wait()

        startg(0, 0)
        startg(1, 1)

        def group(g, carry):
            for b in range(2):
                ci = g * 2 + b
                waitg(ci, b)
                pltpu.sync_copy(bufs[b], o_hbm.at[pl.ds(base + ci * rb, rb)])
                nci = ci + 2

                @pl.when(nci < nch)
                def _():
                    startg(nci, b)
            return carry

        lax.fori_loop(0, nch // 2, group, 0)

    return k(table, idx)


def _sc_segment_sum(msg, adjc, np_rows):
    ep = msg.shape[0]
    nchunks = adjc.shape[0] // ep
    assert nchunks % 2 == 0
    cps = nchunks // 2
    ept = ep // 16
    nch = ept // _CHS
    assert ept % _CHS == 0 and nch % 2 == 0
    tpr = _C // 16
    nz = tpr // _CH
    assert tpr % _CH == 0

    scratch = [
        pltpu.VMEM((_CHS, 128), jnp.float32),
        pltpu.VMEM((_CHS, 128), jnp.float32),
        pltpu.VMEM((_CHS,), jnp.int32),
        pltpu.VMEM((_CHS,), jnp.int32),
        pltpu.VMEM_SHARED((_C + 8, 128), jnp.float32),
        pltpu.SemaphoreType.DMA,
        pltpu.SemaphoreType.DMA,
    ]

    @functools.partial(
        pl.kernel, mesh=_sc_mesh(),
        out_type=jax.ShapeDtypeStruct((np_rows, 128), jnp.float32),
        scratch_types=scratch)
    def k(msg_hbm, adj_hbm, zref_hbm, out_hbm, mb0, mb1, ib0, ib1,
          shared, sem0, sem1):
        c = lax.axis_index("c")
        s = lax.axis_index("s")
        tbase = s * ept
        mbufs = (mb0, mb1)
        ibufs = (ib0, ib1)
        sems = (sem0, sem1)

        for jj in range(cps):
            chunk = c * cps + jj
            abase = chunk * ep + tbase

            def zs(z, carry):
                pltpu.sync_copy(zref_hbm,
                                shared.at[pl.ds(s * tpr + z * _CH, _CH)])
                return carry

            lax.fori_loop(0, nz, zs, 0)

            @pl.when(s == 0)
            def _():
                pltpu.sync_copy(zref_hbm.at[pl.ds(0, 8)],
                                shared.at[pl.ds(_C, 8)])

            plsc.subcore_barrier()

            def start(ci, b):
                pltpu.async_copy(msg_hbm.at[pl.ds(tbase + ci * _CHS, _CHS)],
                                 mbufs[b], sems[b])
                pltpu.async_copy(adj_hbm.at[pl.ds(abase + ci * _CHS, _CHS)],
                                 ibufs[b], sems[b])

            def wait(ci, b):
                pltpu.make_async_copy(
                    msg_hbm.at[pl.ds(tbase + ci * _CHS, _CHS)], mbufs[b],
                    sems[b]).wait()
                pltpu.make_async_copy(
                    adj_hbm.at[pl.ds(abase + ci * _CHS, _CHS)], ibufs[b],
                    sems[b]).wait()

            start(0, 0)
            start(1, 1)

            def group(g, carry):
                for b in range(2):
                    ci = g * 2 + b
                    wait(ci, b)
                    pltpu.sync_copy(mbufs[b], shared.at[ibufs[b]], add=True)
                    nci = ci + 2

                    @pl.when(nci < nch)
                    def _():
                        start(nci, b)
                return carry

            lax.fori_loop(0, nch // 2, group, 0)
            plsc.subcore_barrier()

            def co(z, carry):
                r0 = s * tpr + z * _CHS
                g0 = chunk * _C + r0

                @pl.when(g0 + _CHS <= np_rows)
                def _():
                    pltpu.sync_copy(shared.at[pl.ds(r0, _CHS)], mb0)
                    pltpu.sync_copy(mb0, out_hbm.at[pl.ds(g0, _CHS)])
                return carry

            lax.fori_loop(0, tpr // _CHS, co, 0)
            plsc.subcore_barrier()

    return k(msg, adjc, jnp.zeros((_CH, 128), jnp.float32))


def _scatter_rows(idx, np_rows):
    nchunks = -(-np_rows // _C)
    nchunks += nchunks % 2
    rows = [jnp.where((idx >= j * _C) & (idx < (j + 1) * _C), idx - j * _C, _C)
            for j in range(nchunks)]
    return jnp.concatenate(rows).astype(jnp.int32)


def kernel(grid_x, grid_struct, mesh_struct, g2m_edge_attr, m2m_edge_attr,
           m2g_edge_attr, g2m_senders, g2m_receivers, m2m_senders,
           m2m_receivers, m2g_senders, m2g_receivers, params):
    p = params
    gx = _pad_rows(grid_x, _NGP)
    gs = jnp.pad(grid_struct, ((0, _NGP - _NG), (0, 5)))
    ms = jnp.pad(mesh_struct, ((0, _NMP - _NM), (0, 5)))
    ea_g = jnp.pad(g2m_edge_attr, ((0, _EGP - _EG), (0, 4)))
    ea_m = jnp.pad(m2m_edge_attr, ((0, _EMP - _EM), (0, 4)))
    ea_d = jnp.pad(m2g_edge_attr, ((0, _EDP - _ED), (0, 4)))

    sg = _pad_idx(g2m_senders, _EGP, 0)
    rg_g = _pad_idx(g2m_receivers, _EGP, 0)
    rg_s = _scatter_rows(_pad_idx(g2m_receivers, _EGP, _BIGI), _NMP)
    sm = _pad_idx(m2m_senders, _EMP, 0)
    rm_g = _pad_idx(m2m_receivers, _EMP, 0)
    rm_s = _scatter_rows(_pad_idx(m2m_receivers, _EMP, _BIGI), _NMP)
    sd = _pad_idx(m2g_senders, _EDP, 0)
    rd_g = _pad_idx(m2g_receivers, _EDP, 0)
    rd_s = _scatter_rows(_pad_idx(m2g_receivers, _EDP, _BIGI), _NGP)

    mp = p["grid_embed"]
    w1 = mp["w1"]
    grid = _tc_mlp([gx, gs], [w1[:128], jnp.pad(w1[128:], ((0, 5), (0, 0)))], mp)
    mp = p["mesh_embed"]
    mesh = _tc_mlp([ms], [jnp.pad(mp["w1"], ((0, 5), (0, 0)))], mp)
    mp = p["g2m_edge_embed"]
    eg = _tc_mlp([ea_g], [jnp.pad(mp["w1"], ((0, 4), (0, 0)))], mp)
    mp = p["m2m_edge_embed"]
    em = _tc_mlp([ea_m], [jnp.pad(mp["w1"], ((0, 4), (0, 0)))], mp)
    mp = p["m2g_edge_embed"]
    ed = _tc_mlp([ea_d], [jnp.pad(mp["w1"], ((0, 4), (0, 0)))], mp)

    snd = _sc_gather(grid, sg)
    rcv = _sc_gather(mesh, rg_g)
    mp = p["g2m_edge"]
    w1 = mp["w1"]
    msg = _tc_mlp([eg, snd, rcv], [w1[:128], w1[128:256], w1[256:384]], mp)
    agg = _sc_segment_sum(msg, rg_s, _NMP)
    mp = p["g2m_node"]
    w1 = mp["w1"]
    mesh = _tc_mlp([mesh, agg], [w1[:128], w1[128:256]], mp, res=mesh)
    mp = p["g2m_grid"]
    grid = _tc_mlp([grid], [mp["w1"]], mp, res=grid)

    for pe, pn in zip(p["proc_edge"], p["proc_node"]):
        snd = _sc_gather(mesh, sm)
        rcv = _sc_gather(mesh, rm_g)
        w1 = pe["w1"]
        em = _tc_mlp([em, snd, rcv], [w1[:128], w1[128:256], w1[256:384]], pe,
                     res=em)
        agg = _sc_segment_sum(em, rm_s, _NMP)
        w1 = pn["w1"]
        mesh = _tc_mlp([mesh, agg], [w1[:128], w1[128:256]], pn, res=mesh)

    snd = _sc_gather(mesh, sd)
    rcv = _sc_gather(grid, rd_g)
    mp = p["m2g_edge"]
    w1 = mp["w1"]
    msg = _tc_mlp([ed, snd, rcv], [w1[:128], w1[128:256], w1[256:384]], mp)
    agg = _sc_segment_sum(msg, rd_s, _NGP)
    mp = p["m2g_node"]
    w1 = mp["w1"]
    grid = _tc_mlp([grid, agg], [w1[:128], w1[128:256]], mp, res=grid)
    mp = p["out"]
    out = _tc_mlp([grid], [mp["w1"]], mp, ln=False, res=gx)
    return out[:_NG]


# speedup vs baseline: 1.3188x; 1.2823x over previous
import functools

import jax
import jax.numpy as jnp
from jax import lax
from jax.experimental import pallas as pl
from jax.experimental.pallas import tpu as pltpu
from jax.experimental.pallas import tpu_sc as plsc

_NG, _NM = 65160, 40962
_NGP, _NMP = 65536, 41472
_EG, _EM, _ED = 300000, 250000, 195480
_EGP, _EMP, _EDP = 303104, 253952, 196608
_BIGI = 1 << 29
_C = 12288
_CH = 128
_CHS = 64


def _pad_rows(x, n):
    return jnp.pad(x, ((0, n - x.shape[0]), (0, 0)))


def _pad_idx(x, n, fill):
    return jnp.pad(x.astype(jnp.int32), (0, n - x.shape[0]), constant_values=fill)


def _tc_mlp(xs, w1s, mp, ln=True, res=None, block=512):
    rows = xs[0].shape[0]
    assert rows % block == 0, (rows, block)
    nx = len(xs)
    nres = 1 if res is not None else 0
    b1 = mp["b1"].reshape(1, -1)
    b2 = mp["b2"].reshape(1, -1)
    wparams = [b1, mp["w2"], b2]
    if ln:
        wparams += [mp["g"].reshape(1, -1), mp["bln"].reshape(1, -1)]
    ops = list(xs) + list(w1s) + wparams + ([res] if res is not None else [])
    dout = mp["w2"].shape[1]

    def body(*refs):
        xr = refs[:nx]
        w1r = refs[nx:2 * nx]
        pos = 2 * nx
        b1_r, w2_r, b2_r = refs[pos], refs[pos + 1], refs[pos + 2]
        pos += 3
        if ln:
            g_r, bln_r = refs[pos], refs[pos + 1]
            pos += 2
        res_r = refs[pos:pos + nres]
        out_r = refs[pos + nres]
        h = jnp.dot(xr[0][...], w1r[0][...], preferred_element_type=jnp.float32)
        for i in range(1, nx):
            h = h + jnp.dot(xr[i][...], w1r[i][...],
                            preferred_element_type=jnp.float32)
        h = h + b1_r[...]
        h = h * jax.nn.sigmoid(h)
        o = jnp.dot(h, w2_r[...], preferred_element_type=jnp.float32) + b2_r[...]
        if ln:
            m = jnp.mean(o, axis=-1, keepdims=True)
            v = jnp.mean((o - m) * (o - m), axis=-1, keepdims=True)
            o = (o - m) * lax.rsqrt(v + 1e-5) * g_r[...] + bln_r[...]
        if nres:
            o = o + res_r[0][...]
        out_r[...] = o

    in_specs = []
    for x in xs:
        in_specs.append(pl.BlockSpec((block, x.shape[1]), lambda i: (i, 0)))
    for w in list(w1s) + wparams:
        in_specs.append(pl.BlockSpec(w.shape, lambda i: (0, 0)))
    if res is not None:
        in_specs.append(pl.BlockSpec((block, res.shape[1]), lambda i: (i, 0)))
    out = pl.pallas_call(
        body,
        grid=(rows // block,),
        in_specs=in_specs,
        out_specs=pl.BlockSpec((block, dout), lambda i: (i, 0)),
        out_shape=jax.ShapeDtypeStruct((rows, dout), jnp.float32),
    )(*ops)
    return out


def _sc_mesh():
    return plsc.VectorSubcoreMesh(core_axis_name="c", subcore_axis_name="s")


def _sc_gather(table, idx):
    ep = idx.shape[0]
    d = table.shape[1]
    bpw = ep // 32
    nch = bpw // _CH
    assert bpw % _CH == 0 and nch % 2 == 0

    @functools.partial(
        pl.kernel,
        mesh=_sc_mesh(),
        out_type=jax.ShapeDtypeStruct((ep, d), jnp.float32),
        scratch_types=[
            pltpu.VMEM((bpw,), jnp.int32),
            pltpu.VMEM((_CH, d), jnp.float32),
            pltpu.VMEM((_CH, d), jnp.float32),
            pltpu.SemaphoreType.DMA,
            pltpu.SemaphoreType.DMA,
        ],
    )
    def k(table_hbm, idx_hbm, out_hbm, idx_v, buf0, buf1, sem0, sem1):
        wid = lax.axis_index("s") * 2 + lax.axis_index("c")
        base = wid * bpw
        pltpu.sync_copy(idx_hbm.at[pl.ds(base, bpw)], idx_v)
        bufs = (buf0, buf1)
        sems = (sem0, sem1)

        def start(ci, b):
            pltpu.async_copy(
                table_hbm.at[idx_v.at[pl.ds(ci * _CH, _CH)]], bufs[b], sems[b])

        def wait(ci, b):
            pltpu.make_async_copy(
                table_hbm.at[idx_v.at[pl.ds(ci * _CH, _CH)]], bufs[b],
                sems[b]).wait()

        start(0, 0)
        start(1, 1)

        def group(g, carry):
            for b in range(2):
                ci = g * 2 + b
                wait(ci, b)
                pltpu.sync_copy(bufs[b], out_hbm.at[pl.ds(base + ci * _CH, _CH)])
                nci = ci + 2

                @pl.when(nci < nch)
                def _():
                    start(nci, b)
            return carry

        lax.fori_loop(0, nch // 2, group, 0)

    return k(table, idx)


def _sc_segment_sum(msg, adjc, np_rows):
    ep = msg.shape[0]
    nchunks = adjc.shape[0] // ep
    assert nchunks % 2 == 0
    cps = nchunks // 2
    ept = ep // 16
    nch = ept // _CHS
    assert ept % _CHS == 0 and nch % 2 == 0
    tpr = _C // 16
    nz = tpr // _CH
    assert tpr % _CH == 0

    scratch = [
        pltpu.VMEM((_CHS, 128), jnp.float32),
        pltpu.VMEM((_CHS, 128), jnp.float32),
        pltpu.VMEM((_CHS,), jnp.int32),
        pltpu.VMEM((_CHS,), jnp.int32),
        pltpu.VMEM_SHARED((_C + 8, 128), jnp.float32),
        pltpu.SemaphoreType.DMA,
        pltpu.SemaphoreType.DMA,
    ]

    @functools.partial(
        pl.kernel, mesh=_sc_mesh(),
        out_type=jax.ShapeDtypeStruct((np_rows, 128), jnp.float32),
        scratch_types=scratch)
    def k(msg_hbm, adj_hbm, zref_hbm, out_hbm, mb0, mb1, ib0, ib1,
          shared, sem0, sem1):
        c = lax.axis_index("c")
        s = lax.axis_index("s")
        tbase = s * ept
        mbufs = (mb0, mb1)
        ibufs = (ib0, ib1)
        sems = (sem0, sem1)

        for jj in range(cps):
            chunk = c * cps + jj
            abase = chunk * ep + tbase

            def zs(z, carry):
                pltpu.sync_copy(zref_hbm,
                                shared.at[pl.ds(s * tpr + z * _CH, _CH)])
                return carry

            lax.fori_loop(0, nz, zs, 0)

            @pl.when(s == 0)
            def _():
                pltpu.sync_copy(zref_hbm.at[pl.ds(0, 8)],
                                shared.at[pl.ds(_C, 8)])

            plsc.subcore_barrier()

            def start(ci, b):
                pltpu.async_copy(msg_hbm.at[pl.ds(tbase + ci * _CHS, _CHS)],
                                 mbufs[b], sems[b])
                pltpu.async_copy(adj_hbm.at[pl.ds(abase + ci * _CHS, _CHS)],
                                 ibufs[b], sems[b])

            def wait(ci, b):
                pltpu.make_async_copy(
                    msg_hbm.at[pl.ds(tbase + ci * _CHS, _CHS)], mbufs[b],
                    sems[b]).wait()
                pltpu.make_async_copy(
                    adj_hbm.at[pl.ds(abase + ci * _CHS, _CHS)], ibufs[b],
                    sems[b]).wait()

            start(0, 0)
            start(1, 1)

            def group(g, carry):
                for b in range(2):
                    ci = g * 2 + b
                    wait(ci, b)
                    pltpu.sync_copy(mbufs[b], shared.at[ibufs[b]], add=True)
                    nci = ci + 2

                    @pl.when(nci < nch)
                    def _():
                        start(nci, b)
                return carry

            lax.fori_loop(0, nch // 2, group, 0)
            plsc.subcore_barrier()

            def co(z, carry):
                r0 = s * tpr + z * _CHS
                g0 = chunk * _C + r0

                @pl.when(g0 + _CHS <= np_rows)
                def _():
                    pltpu.sync_copy(shared.at[pl.ds(r0, _CHS)], mb0)
                    pltpu.sync_copy(mb0, out_hbm.at[pl.ds(g0, _CHS)])
                return carry

            lax.fori_loop(0, tpr // _CHS, co, 0)
            plsc.subcore_barrier()

    return k(msg, adjc, jnp.zeros((_CH, 128), jnp.float32))


def _scatter_rows(idx, np_rows):
    nchunks = -(-np_rows // _C)
    nchunks += nchunks % 2
    rows = [jnp.where((idx >= j * _C) & (idx < (j + 1) * _C), idx - j * _C, _C)
            for j in range(nchunks)]
    return jnp.concatenate(rows).astype(jnp.int32)


def kernel(grid_x, grid_struct, mesh_struct, g2m_edge_attr, m2m_edge_attr,
           m2g_edge_attr, g2m_senders, g2m_receivers, m2m_senders,
           m2m_receivers, m2g_senders, m2g_receivers, params):
    p = params
    gx = _pad_rows(grid_x, _NGP)
    gs = jnp.pad(grid_struct, ((0, _NGP - _NG), (0, 5)))
    ms = jnp.pad(mesh_struct, ((0, _NMP - _NM), (0, 5)))
    ea_g = jnp.pad(g2m_edge_attr, ((0, _EGP - _EG), (0, 4)))
    ea_m = jnp.pad(m2m_edge_attr, ((0, _EMP - _EM), (0, 4)))
    ea_d = jnp.pad(m2g_edge_attr, ((0, _EDP - _ED), (0, 4)))

    sg = _pad_idx(g2m_senders, _EGP, 0)
    rg_g = _pad_idx(g2m_receivers, _EGP, 0)
    rg_s = _scatter_rows(_pad_idx(g2m_receivers, _EGP, _BIGI), _NMP)
    sm = _pad_idx(m2m_senders, _EMP, 0)
    rm_g = _pad_idx(m2m_receivers, _EMP, 0)
    rm_s = _scatter_rows(_pad_idx(m2m_receivers, _EMP, _BIGI), _NMP)
    sd = _pad_idx(m2g_senders, _EDP, 0)
    rd_g = _pad_idx(m2g_receivers, _EDP, 0)
    rd_s = _scatter_rows(_pad_idx(m2g_receivers, _EDP, _BIGI), _NGP)

    mp = p["grid_embed"]
    w1 = mp["w1"]
    grid = _tc_mlp([gx, gs], [w1[:128], jnp.pad(w1[128:], ((0, 5), (0, 0)))], mp)
    mp = p["mesh_embed"]
    mesh = _tc_mlp([ms], [jnp.pad(mp["w1"], ((0, 5), (0, 0)))], mp)
    mp = p["g2m_edge_embed"]
    eg = _tc_mlp([ea_g], [jnp.pad(mp["w1"], ((0, 4), (0, 0)))], mp)
    mp = p["m2m_edge_embed"]
    em = _tc_mlp([ea_m], [jnp.pad(mp["w1"], ((0, 4), (0, 0)))], mp)
    mp = p["m2g_edge_embed"]
    ed = _tc_mlp([ea_d], [jnp.pad(mp["w1"], ((0, 4), (0, 0)))], mp)

    snd = _sc_gather(grid, sg)
    rcv = _sc_gather(mesh, rg_g)
    mp = p["g2m_edge"]
    w1 = mp["w1"]
    msg = _tc_mlp([eg, snd, rcv], [w1[:128], w1[128:256], w1[256:384]], mp)
    agg = _sc_segment_sum(msg, rg_s, _NMP)
    mp = p["g2m_node"]
    w1 = mp["w1"]
    mesh = _tc_mlp([mesh, agg], [w1[:128], w1[128:256]], mp, res=mesh)
    mp = p["g2m_grid"]
    grid = _tc_mlp([grid], [mp["w1"]], mp, res=grid)

    for pe, pn in zip(p["proc_edge"], p["proc_node"]):
        snd = _sc_gather(mesh, sm)
        rcv = _sc_gather(mesh, rm_g)
        w1 = pe["w1"]
        em = _tc_mlp([em, snd, rcv], [w1[:128], w1[128:256], w1[256:384]], pe,
                     res=em)
        agg = _sc_segment_sum(em, rm_s, _NMP)
        w1 = pn["w1"]
        mesh = _tc_mlp([mesh, agg], [w1[:128], w1[128:256]], pn, res=mesh)

    snd = _sc_gather(mesh, sd)
    rcv = _sc_gather(grid, rd_g)
    mp = p["m2g_edge"]
    w1 = mp["w1"]
    msg = _tc_mlp([ed, snd, rcv], [w1[:128], w1[128:256], w1[256:384]], mp)
    agg = _sc_segment_sum(msg, rd_s, _NGP)
    mp = p["m2g_node"]
    w1 = mp["w1"]
    grid = _tc_mlp([grid, agg], [w1[:128], w1[128:256]], mp, res=grid)
    mp = p["out"]
    out = _tc_mlp([grid], [mp["w1"]], mp, ln=False, res=gx)
    return out[:_NG]
